# baseline (device time: 10234 ns/iter reference)
import jax
import jax.numpy as jnp
from jax import lax
from jax.experimental import pallas as pl
from jax.experimental.pallas import tpu as pltpu

N_CHUNKS = 2

_SEM_A = 0
_SEM_SCALES = 1
_SEM_B0 = 2
_N_SEMS = 2 + N_CHUNKS

_SCALES_W = 128


def _quantize(x, inv_scale):
    q = jnp.clip(x * (1.0 / inv_scale), -127.0, 127.0)
    q = q + jnp.where(q >= 0.0, 0.5, -0.5)
    return q.astype(jnp.int8)


def kernel(A, B):
    m, k = A.shape
    _, n = B.shape
    nc = n // N_CHUNKS

    def body(a_ref, b_ref, out_ref, a_bf, b_bf, a_q, b_q, a_rcv, b_rcv,
             a_rcv_bf, scales, scales_rcv, send_sems, recv_sems):
        my_x = lax.axis_index("x")
        my_y = lax.axis_index("y")
        partner = (1 - my_x, my_y)

        barrier_sem = pltpu.get_barrier_semaphore()
        pl.semaphore_signal(
            barrier_sem, inc=1,
            device_id=partner, device_id_type=pl.DeviceIdType.MESH,
        )

        inv_a = jnp.maximum(jnp.max(jnp.abs(a_ref[...])), 1e-20) / 127.0
        inv_b = [
            jnp.maximum(
                jnp.max(jnp.abs(b_ref[:, pl.ds(j * nc, nc)])), 1e-20
            ) / 127.0
            for j in range(N_CHUNKS)
        ]
        lane = lax.broadcasted_iota(jnp.int32, (1, _SCALES_W), 1)
        row = jnp.where(lane == _SEM_A - _SEM_A, inv_a, 0.0)
        for j in range(N_CHUNKS):
            row = jnp.where(lane == 1 + j, inv_b[j], row)
        scales[...] = row

        a_q[...] = _quantize(a_ref[...], inv_a)

        pl.semaphore_wait(barrier_sem, 1)

        rdma_a = pltpu.make_async_remote_copy(
            src_ref=a_q, dst_ref=a_rcv,
            send_sem=send_sems.at[_SEM_A], recv_sem=recv_sems.at[_SEM_A],
            device_id=partner, device_id_type=pl.DeviceIdType.MESH,
        )
        rdma_a.start()
        rdma_s = pltpu.make_async_remote_copy(
            src_ref=scales, dst_ref=scales_rcv,
            send_sem=send_sems.at[_SEM_SCALES],
            recv_sem=recv_sems.at[_SEM_SCALES],
            device_id=partner, device_id_type=pl.DeviceIdType.MESH,
        )
        rdma_s.start()
        rdma_bs = []
        for j in range(N_CHUNKS):
            b_q[j] = _quantize(b_ref[:, pl.ds(j * nc, nc)], inv_b[j])
            r = pltpu.make_async_remote_copy(
                src_ref=b_q.at[j], dst_ref=b_rcv.at[j],
                send_sem=send_sems.at[_SEM_B0 + j],
                recv_sem=recv_sems.at[_SEM_B0 + j],
                device_id=partner, device_id_type=pl.DeviceIdType.MESH,
            )
            r.start()
            rdma_bs.append(r)

        a_bf[...] = a_ref[...].astype(jnp.bfloat16)
        for j in range(N_CHUNKS):
            b_bf[j] = b_ref[:, pl.ds(j * nc, nc)].astype(jnp.bfloat16)
            out_ref[:, pl.ds(j * nc, nc)] = jnp.dot(
                a_bf[...], b_bf[j], preferred_element_type=jnp.float32
            ).astype(jnp.bfloat16)

        rdma_a.wait_recv()
        rdma_s.wait_recv()
        a_rcv_bf[...] = a_rcv[...].astype(jnp.bfloat16)
        inv_a_r = scales_rcv[0, 0]
        for j in range(N_CHUNKS):
            rdma_bs[j].wait_recv()
            out_ref[:, pl.ds(j * nc, nc)] += (
                jnp.dot(
                    a_rcv_bf[...], b_rcv[j].astype(jnp.bfloat16),
                    preferred_element_type=jnp.float32,
                ) * (inv_a_r * scales_rcv[0, 1 + j])
            ).astype(jnp.bfloat16)

        rdma_a.wait_send()
        rdma_s.wait_send()
        for j in range(N_CHUNKS):
            rdma_bs[j].wait_send()

    return pl.pallas_call(
        body,
        out_shape=jax.ShapeDtypeStruct((m, n), jnp.bfloat16),
        in_specs=[
            pl.BlockSpec(memory_space=pltpu.MemorySpace.VMEM),
            pl.BlockSpec(memory_space=pltpu.MemorySpace.VMEM),
        ],
        out_specs=pl.BlockSpec(memory_space=pltpu.MemorySpace.VMEM),
        scratch_shapes=[
            pltpu.VMEM((m, k), jnp.bfloat16),
            pltpu.VMEM((N_CHUNKS, k, nc), jnp.bfloat16),
            pltpu.VMEM((m, k), jnp.int8),
            pltpu.VMEM((N_CHUNKS, k, nc), jnp.int8),
            pltpu.VMEM((m, k), jnp.int8),
            pltpu.VMEM((N_CHUNKS, k, nc), jnp.int8),
            pltpu.VMEM((m, k), jnp.bfloat16),
            pltpu.VMEM((1, _SCALES_W), jnp.float32),
            pltpu.VMEM((1, _SCALES_W), jnp.float32),
            pltpu.SemaphoreType.DMA((_N_SEMS,)),
            pltpu.SemaphoreType.DMA((_N_SEMS,)),
        ],
        compiler_params=pltpu.CompilerParams(collective_id=0),
    )(A, B)
